# Initial kernel scaffold; baseline (speedup 1.0000x reference)
import jax
import jax.numpy as jnp
from jax.experimental import pallas as pl


def kernel(skill_embs, edge_index, W1, b1, W2, b2, W3, b3):
    raise NotImplementedError("write your pallas kernel here")


# speedup vs baseline: 10.6575x; 10.6575x over previous
import functools

import jax
import jax.numpy as jnp
from jax import lax
from jax.experimental import pallas as pl
from jax.experimental.pallas import tpu as pltpu
from jax.experimental.pallas import tpu_sc as plsc

N = 10000
D = 128
E = 320000

NC = 2
NS = 16
NW = NC * NS
CHUNK = 128
EPT = -(-(E // NW) // CHUNK) * CHUNK
CPT = EPT // CHUNK
E_PAD = EPT * NW
N_PAD = 10240
RPT = N_PAD // NS

B = 1000
GRID = N // B

_mesh = plsc.VectorSubcoreMesh(
    core_axis_name="c", subcore_axis_name="s", num_cores=NC, num_subcores=NS
)


@functools.partial(
    pl.kernel,
    out_type=jax.ShapeDtypeStruct((NC * N_PAD, D), jnp.float32),
    mesh=_mesh,
    scratch_types=[
        pltpu.VMEM((CPT, CHUNK), jnp.int32),
        pltpu.VMEM((CPT, CHUNK), jnp.int32),
        pltpu.VMEM((CHUNK, D), jnp.float32),
        pltpu.VMEM_SHARED((N_PAD, D), jnp.float32),
    ],
)
def _sc_agg(y_hbm, src_hbm, dst_hbm, out_hbm, sidx, didx, buf, acc):
    c = lax.axis_index("c")
    s = lax.axis_index("s")
    wid = s * NC + c

    pltpu.sync_copy(src_hbm.at[wid], sidx)
    pltpu.sync_copy(dst_hbm.at[wid], didx)

    def _zrow(r, carry):
        for k in range(D // 16):
            buf[r, pl.ds(k * 16, 16)] = jnp.zeros((16,), jnp.float32)
        return carry

    lax.fori_loop(0, CHUNK, _zrow, 0)
    for k in range(RPT // CHUNK):
        pltpu.sync_copy(buf, acc.at[pl.ds(s * RPT + k * CHUNK, CHUNK)])
    plsc.subcore_barrier()

    def _edge_chunk(j, carry):
        pltpu.sync_copy(y_hbm.at[sidx.at[j]], buf)
        pltpu.sync_copy(buf, acc.at[didx.at[j]], add=True)
        return carry

    lax.fori_loop(0, CPT, _edge_chunk, 0)
    plsc.subcore_barrier()
    pltpu.sync_copy(
        acc.at[pl.ds(s * RPT, RPT)],
        out_hbm.at[pl.ds(c * N_PAD + s * RPT, RPT)],
    )


@functools.partial(
    pl.kernel,
    out_type=jax.ShapeDtypeStruct((NC * N_PAD, 16), jnp.float32),
    mesh=_mesh,
    scratch_types=[
        pltpu.VMEM((CPT, CHUNK), jnp.int32),
        pltpu.VMEM((CHUNK, 16), jnp.float32),
        pltpu.VMEM_SHARED((N_PAD, 16), jnp.float32),
    ],
)
def _sc_deg(dst_hbm, out_hbm, didx, obuf, acc):
    c = lax.axis_index("c")
    s = lax.axis_index("s")
    wid = s * NC + c

    pltpu.sync_copy(dst_hbm.at[wid], didx)

    def _zrow(r, carry):
        obuf[r, :] = jnp.zeros((16,), jnp.float32)
        return carry

    lax.fori_loop(0, CHUNK, _zrow, 0)
    for k in range(RPT // CHUNK):
        pltpu.sync_copy(obuf, acc.at[pl.ds(s * RPT + k * CHUNK, CHUNK)])

    def _orow(r, carry):
        obuf[r, :] = jnp.ones((16,), jnp.float32)
        return carry

    lax.fori_loop(0, CHUNK, _orow, 0)
    plsc.subcore_barrier()

    def _edge_chunk(j, carry):
        pltpu.sync_copy(obuf, acc.at[didx.at[j]], add=True)
        return carry

    lax.fori_loop(0, CPT, _edge_chunk, 0)
    plsc.subcore_barrier()
    pltpu.sync_copy(
        acc.at[pl.ds(s * RPT, RPT)],
        out_hbm.at[pl.ds(c * N_PAD + s * RPT, RPT)],
    )


def _tc_prep_body(h0_ref, h1_ref, x_ref, w_ref, dv_ref, y_ref):
    deg = 1.0 + h0_ref[:, 0:1] + h1_ref[:, 0:1]
    dinv = lax.rsqrt(deg)
    dv_ref[...] = jnp.broadcast_to(dinv, (dinv.shape[0], D))
    y_ref[...] = dinv * jnp.dot(
        x_ref[...], w_ref[...], preferred_element_type=jnp.float32
    )


def _tc_mid_body(a0_ref, a1_ref, yp_ref, dv_ref, b_ref, w_ref, yn_ref):
    x = jnp.maximum(
        dv_ref[...] * (a0_ref[...] + a1_ref[...] + yp_ref[...]) + b_ref[...], 0.0
    )
    yn_ref[...] = dv_ref[...] * jnp.dot(
        x, w_ref[...], preferred_element_type=jnp.float32
    )


def _tc_fin_body(a0_ref, a1_ref, yp_ref, dv_ref, b_ref, out_ref):
    out_ref[...] = jnp.maximum(
        dv_ref[...] * (a0_ref[...] + a1_ref[...] + yp_ref[...]) + b_ref[...], 0.0
    )


_row_spec = pl.BlockSpec((B, D), lambda i: (i, 0))
_h_spec = pl.BlockSpec((B, 16), lambda i: (i, 0))
_w_spec = pl.BlockSpec((D, D), lambda i: (0, 0))
_b_spec = pl.BlockSpec((1, D), lambda i: (0, 0))

_tc_prep = pl.pallas_call(
    _tc_prep_body,
    grid=(GRID,),
    in_specs=[_h_spec, _h_spec, _row_spec, _w_spec],
    out_specs=[_row_spec, _row_spec],
    out_shape=[jax.ShapeDtypeStruct((N, D), jnp.float32)] * 2,
)

_tc_mid = pl.pallas_call(
    _tc_mid_body,
    grid=(GRID,),
    in_specs=[_row_spec, _row_spec, _row_spec, _row_spec, _b_spec, _w_spec],
    out_specs=_row_spec,
    out_shape=jax.ShapeDtypeStruct((N, D), jnp.float32),
)

_tc_fin = pl.pallas_call(
    _tc_fin_body,
    grid=(GRID,),
    in_specs=[_row_spec, _row_spec, _row_spec, _row_spec, _b_spec],
    out_specs=_row_spec,
    out_shape=jax.ShapeDtypeStruct((N, D), jnp.float32),
)


def kernel(skill_embs, edge_index, W1, b1, W2, b2, W3, b3):
    src = edge_index[0]
    dst = edge_index[1]
    pad = E_PAD - E
    srcw = jnp.concatenate([src, jnp.zeros((pad,), jnp.int32)]).reshape(
        NW, CPT, CHUNK
    )
    dstw = jnp.concatenate([dst, jnp.full((pad,), N, jnp.int32)]).reshape(
        NW, CPT, CHUNK
    )

    hist = _sc_deg(dstw)
    h0 = hist[:N]
    h1 = hist[N_PAD : N_PAD + N]
    dv, y1 = _tc_prep(h0, h1, skill_embs, W1)

    a = _sc_agg(y1, srcw, dstw)
    y2 = _tc_mid(a[:N], a[N_PAD : N_PAD + N], y1, dv, b1.reshape(1, D), W2)

    a = _sc_agg(y2, srcw, dstw)
    y3 = _tc_mid(a[:N], a[N_PAD : N_PAD + N], y2, dv, b2.reshape(1, D), W3)

    a = _sc_agg(y3, srcw, dstw)
    return _tc_fin(a[:N], a[N_PAD : N_PAD + N], y3, dv, b3.reshape(1, D))
